# baseline (speedup 1.0000x reference)
import functools

import jax
import jax.numpy as jnp
from jax import lax
from jax.experimental import pallas as pl
from jax.experimental.pallas import tpu as pltpu
from jax.experimental.pallas import tpu_sc as plsc

N_NODES = 10000
N_EDGES = 160000
D_IN = 256
D_HID = 256
N_CLS = 64

CHUNK = 128
ACHUNK = 64
E_PAD = 163840
N_ACC = 10112
ROWS_T = N_ACC // 16
ROW_BLK = 400
N_GRID = N_NODES // ROW_BLK


@functools.lru_cache(maxsize=None)
def _sc_kernels():
    mesh = plsc.VectorSubcoreMesh(core_axis_name="c", subcore_axis_name="s")
    deg = functools.partial(
        pl.kernel,
        mesh=mesh,
        out_type=jax.ShapeDtypeStruct((2, N_ACC, 128), jnp.float32),
        scratch_types=[
            pltpu.VMEM((80, CHUNK), jnp.int32),
            pltpu.VMEM((CHUNK, 128), jnp.float32),
            pltpu.VMEM_SHARED((N_ACC, 128), jnp.float32),
            pltpu.SemaphoreType.DMA,
        ],
    )(_deg_body)
    agg1 = functools.partial(
        pl.kernel,
        mesh=mesh,
        out_type=jax.ShapeDtypeStruct((2, N_ACC, 128), jnp.float32),
        scratch_types=[
            pltpu.VMEM((40, ACHUNK), jnp.int32),
            pltpu.VMEM((40, ACHUNK), jnp.int32),
            pltpu.VMEM((ACHUNK, 128), jnp.float32),
            pltpu.VMEM((ACHUNK, 128), jnp.float32),
            pltpu.VMEM((ACHUNK, 128), jnp.float32),
            pltpu.VMEM((ACHUNK, 128), jnp.float32),
            pltpu.VMEM_SHARED((N_ACC, 128), jnp.float32),
            pltpu.SemaphoreType.DMA,
            pltpu.SemaphoreType.DMA,
        ],
    )(_agg1_body)
    agg2 = functools.partial(
        pl.kernel,
        mesh=mesh,
        out_type=jax.ShapeDtypeStruct((2, N_ACC, 128), jnp.float32),
        scratch_types=[
            pltpu.VMEM((40, ACHUNK), jnp.int32),
            pltpu.VMEM((40, ACHUNK), jnp.int32),
            pltpu.VMEM((ACHUNK, 128), jnp.float32),
            pltpu.VMEM((ACHUNK, 128), jnp.float32),
            pltpu.VMEM((ACHUNK, 128), jnp.float32),
            pltpu.VMEM((ACHUNK, 128), jnp.float32),
            pltpu.VMEM_SHARED((N_ACC, 128), jnp.float32),
            pltpu.SemaphoreType.DMA,
            pltpu.SemaphoreType.DMA,
        ],
    )(_agg2_body)
    return deg, agg1, agg2


def _deg_body(srcd_hbm, dstp_hbm, ones_hbm, z128_hbm, deg_hbm,
              idx_v, ones_v, acc, sem):
    c = lax.axis_index("c")
    s = lax.axis_index("s")
    rz = s * ROWS_T
    pltpu.sync_copy(z128_hbm.at[pl.ds(rz, ROWS_T)], acc.at[pl.ds(rz, ROWS_T)])
    pltpu.sync_copy(ones_hbm, ones_v)

    @pl.when(c == 0)
    def _():
        pltpu.sync_copy(srcd_hbm.at[s], idx_v)

    @pl.when(c == 1)
    def _():
        pltpu.sync_copy(dstp_hbm.at[s], idx_v)

    plsc.subcore_barrier()

    def body(j, carry):
        @pl.when(j >= 8)
        def _():
            pltpu.make_async_copy(ones_v, acc.at[idx_v.at[0]], sem).wait()

        pltpu.async_copy(ones_v, acc.at[idx_v.at[j]], sem, add=True)
        return carry

    lax.fori_loop(0, 80, body, 0)
    for _ in range(8):
        pltpu.make_async_copy(ones_v, acc.at[idx_v.at[0]], sem).wait()
    plsc.subcore_barrier()

    @pl.when(c == 0)
    def _():
        pltpu.sync_copy(acc.at[pl.ds(rz, ROWS_T)],
                        deg_hbm.at[0, pl.ds(rz, ROWS_T)])

    @pl.when(c == 1)
    def _():
        pltpu.sync_copy(acc.at[pl.ds(rz, ROWS_T)],
                        deg_hbm.at[1, pl.ds(rz, ROWS_T)])


def _agg_pipeline(y_hbm, sidx, didx, bufs, acc, gsem, ssem, nchunks):
    pltpu.async_copy(y_hbm.at[sidx.at[0]], bufs[0], gsem)
    pltpu.async_copy(y_hbm.at[sidx.at[1]], bufs[1], gsem)

    def body(j, carry):
        for b in range(4):
            k = 4 * j + b
            buf = bufs[b]
            nxt = bufs[(b + 2) % 4]
            pltpu.make_async_copy(y_hbm.at[sidx.at[k]], buf, gsem).wait()
            pltpu.async_copy(buf, acc.at[didx.at[k]], ssem, add=True)

            @pl.when(k + 2 < nchunks)
            def _():
                @pl.when(k >= 2)
                def _():
                    pltpu.make_async_copy(nxt, acc.at[didx.at[0]],
                                          ssem).wait()

                pltpu.async_copy(y_hbm.at[sidx.at[k + 2]], nxt, gsem)

        return carry

    lax.fori_loop(0, nchunks // 4, body, 0)
    for _ in range(4):
        pltpu.make_async_copy(bufs[0], acc.at[didx.at[0]], ssem).wait()


def _agg1_body(ya_hbm, yb_hbm, srcg_hbm, dstp_hbm, z128_hbm, out_hbm,
               sidx, didx, b0, b1, b2, b3, acc, gsem, ssem):
    c = lax.axis_index("c")
    s = lax.axis_index("s")
    rz = s * ROWS_T
    pltpu.sync_copy(z128_hbm.at[pl.ds(rz, ROWS_T)], acc.at[pl.ds(rz, ROWS_T)])
    plsc.subcore_barrier()

    def run(y_hbm):
        for h in range(4):
            pltpu.sync_copy(srcg_hbm.at[s, pl.ds(h * 40, 40)], sidx)
            pltpu.sync_copy(dstp_hbm.at[s, pl.ds(h * 40, 40)], didx)
            _agg_pipeline(y_hbm, sidx, didx, (b0, b1, b2, b3), acc,
                          gsem, ssem, 40)

    @pl.when(c == 0)
    def _():
        run(ya_hbm)

    @pl.when(c == 1)
    def _():
        run(yb_hbm)

    plsc.subcore_barrier()

    @pl.when(c == 0)
    def _():
        pltpu.sync_copy(acc.at[pl.ds(rz, ROWS_T)],
                        out_hbm.at[0, pl.ds(rz, ROWS_T)])

    @pl.when(c == 1)
    def _():
        pltpu.sync_copy(acc.at[pl.ds(rz, ROWS_T)],
                        out_hbm.at[1, pl.ds(rz, ROWS_T)])


def _agg2_body(y2_hbm, srcg_hbm, dstp_hbm, z128_hbm, out_hbm,
               sidx, didx, b0, b1, b2, b3, acc, gsem, ssem):
    c = lax.axis_index("c")
    s = lax.axis_index("s")
    rz = s * ROWS_T
    pltpu.sync_copy(z128_hbm.at[pl.ds(rz, ROWS_T)], acc.at[pl.ds(rz, ROWS_T)])

    plsc.subcore_barrier()

    def run(ci):
        for h in range(2):
            pltpu.sync_copy(srcg_hbm.at[ci, s, pl.ds(h * 40, 40)], sidx)
            pltpu.sync_copy(dstp_hbm.at[ci, s, pl.ds(h * 40, 40)], didx)
            _agg_pipeline(y2_hbm, sidx, didx, (b0, b1, b2, b3), acc,
                          gsem, ssem, 40)

    @pl.when(c == 0)
    def _():
        run(0)

    @pl.when(c == 1)
    def _():
        run(1)

    plsc.subcore_barrier()

    @pl.when(c == 0)
    def _():
        pltpu.sync_copy(acc.at[pl.ds(rz, ROWS_T)],
                        out_hbm.at[0, pl.ds(rz, ROWS_T)])

    @pl.when(c == 1)
    def _():
        pltpu.sync_copy(acc.at[pl.ds(rz, ROWS_T)],
                        out_hbm.at[1, pl.ds(rz, ROWS_T)])


def _tc1a_body(x_ref, w_ref, u_ref):
    u_ref[...] = jnp.dot(x_ref[...], w_ref[...],
                         preferred_element_type=jnp.float32)


def _tc1b_body(u_ref, deg_ref, ya_ref, yb_ref):
    ns = lax.rsqrt(jnp.maximum(deg_ref[0, :, 0], 1.0))
    y = u_ref[...] * ns[:, None]
    ya_ref[...] = y[:, :128]
    yb_ref[...] = y[:, 128:]


def _tc2_body(agg_ref, deg_ref, b1_ref, w2_ref, y2_ref):
    a1 = jnp.concatenate([agg_ref[0], agg_ref[1]], axis=1)
    ns = lax.rsqrt(jnp.maximum(deg_ref[0, :, 0], 1.0))
    nd = lax.rsqrt(jnp.maximum(deg_ref[1, :, 0], 1.0))
    g = jnp.maximum(a1 * nd[:, None] + b1_ref[0][None, :], 0.0) * ns[:, None]
    y2 = jnp.dot(g, w2_ref[...], preferred_element_type=jnp.float32)
    y2_ref[...] = jnp.concatenate(
        [y2, jnp.zeros((N_NODES, 128 - N_CLS), jnp.float32)], axis=1)


def _tc3_body(p_ref, deg_ref, b2_ref, out_ref):
    nd = lax.rsqrt(jnp.maximum(deg_ref[1, :, 0], 1.0))
    p = p_ref[0, :, :N_CLS] + p_ref[1, :, :N_CLS]
    out_ref[...] = p * nd[:, None] + b2_ref[0][None, :]


def kernel(x, edge_index, W1, b1, W2, b2):
    src = edge_index[0].astype(jnp.int32)
    dst = edge_index[1].astype(jnp.int32)
    npad = E_PAD - N_EDGES
    src_g = jnp.concatenate(
        [src, jnp.arange(npad, dtype=jnp.int32) % N_NODES])
    dummy = N_NODES + (jnp.arange(npad, dtype=jnp.int32) % (N_ACC - N_NODES))
    src_d = jnp.concatenate([src, dummy])
    dst_p = jnp.concatenate([dst, dummy])

    srcd3 = src_d.reshape(16, 80, CHUNK)
    dstp3 = dst_p.reshape(16, 80, CHUNK)
    srcg3 = src_g.reshape(16, 160, ACHUNK)
    dstp3a = dst_p.reshape(16, 160, ACHUNK)
    srcg4 = src_g.reshape(2, 16, 80, ACHUNK)
    dstp4 = dst_p.reshape(2, 16, 80, ACHUNK)

    ones128 = jnp.zeros((CHUNK, 128), jnp.float32).at[:, 0].set(1.0)
    z128 = jnp.zeros((N_ACC, 128), jnp.float32)

    deg_kernel, agg1_kernel, agg2_kernel = _sc_kernels()
    deg = deg_kernel(srcd3, dstp3, ones128, z128)

    u1 = pl.pallas_call(
        _tc1a_body,
        grid=(1,),
        in_specs=[
            pl.BlockSpec((N_NODES, D_IN), lambda i: (0, 0)),
            pl.BlockSpec((D_IN, D_HID), lambda i: (0, 0)),
        ],
        out_specs=pl.BlockSpec((N_NODES, D_HID), lambda i: (0, 0)),
        out_shape=jax.ShapeDtypeStruct((N_NODES, D_HID), jnp.float32),
    )(x, W1)

    ya, yb = pl.pallas_call(
        _tc1b_body,
        grid=(1,),
        in_specs=[
            pl.BlockSpec((N_NODES, D_HID), lambda i: (0, 0)),
            pl.BlockSpec((2, N_NODES, 128), lambda i: (0, 0, 0)),
        ],
        out_specs=[
            pl.BlockSpec((N_NODES, 128), lambda i: (0, 0)),
            pl.BlockSpec((N_NODES, 128), lambda i: (0, 0)),
        ],
        out_shape=[
            jax.ShapeDtypeStruct((N_NODES, 128), jnp.float32),
            jax.ShapeDtypeStruct((N_NODES, 128), jnp.float32),
        ],
    )(u1, deg)

    agg1 = agg1_kernel(ya, yb, srcg3, dstp3a, z128)

    y2 = pl.pallas_call(
        _tc2_body,
        grid=(1,),
        in_specs=[
            pl.BlockSpec((2, N_NODES, 128), lambda i: (0, 0, 0)),
            pl.BlockSpec((2, N_NODES, 128), lambda i: (0, 0, 0)),
            pl.BlockSpec((1, D_HID), lambda i: (0, 0)),
            pl.BlockSpec((D_HID, N_CLS), lambda i: (0, 0)),
        ],
        out_specs=pl.BlockSpec((N_NODES, 128), lambda i: (0, 0)),
        out_shape=jax.ShapeDtypeStruct((N_NODES, 128), jnp.float32),
    )(agg1, deg, b1.reshape(1, D_HID), W2)

    p2 = agg2_kernel(y2, srcg4, dstp4, z128)

    out = pl.pallas_call(
        _tc3_body,
        grid=(1,),
        in_specs=[
            pl.BlockSpec((2, N_NODES, 128), lambda i: (0, 0, 0)),
            pl.BlockSpec((2, N_NODES, 128), lambda i: (0, 0, 0)),
            pl.BlockSpec((1, N_CLS), lambda i: (0, 0)),
        ],
        out_specs=pl.BlockSpec((N_NODES, N_CLS), lambda i: (0, 0)),
        out_shape=jax.ShapeDtypeStruct((N_NODES, N_CLS), jnp.float32),
    )(p2, deg, b2.reshape(1, N_CLS))

    return out


# speedup vs baseline: 1.0099x; 1.0099x over previous
import functools

import jax
import jax.numpy as jnp
from jax import lax
from jax.experimental import pallas as pl
from jax.experimental.pallas import tpu as pltpu
from jax.experimental.pallas import tpu_sc as plsc

N_NODES = 10000
N_EDGES = 160000
D_IN = 256
D_HID = 256
N_CLS = 64

CHUNK = 128
ACHUNK = 64
E_PAD = 163840
N_ACC = 10112
ROWS_T = N_ACC // 16
ROW_BLK = 400
N_GRID = N_NODES // ROW_BLK


@functools.lru_cache(maxsize=None)
def _sc_kernels():
    mesh = plsc.VectorSubcoreMesh(core_axis_name="c", subcore_axis_name="s")
    deg = functools.partial(
        pl.kernel,
        mesh=mesh,
        out_type=jax.ShapeDtypeStruct((2, N_ACC, 128), jnp.float32),
        scratch_types=[
            pltpu.VMEM((80, CHUNK), jnp.int32),
            pltpu.VMEM((CHUNK, 128), jnp.float32),
            pltpu.VMEM_SHARED((N_ACC, 128), jnp.float32),
            pltpu.SemaphoreType.DMA,
        ],
    )(_deg_body)
    agg1 = functools.partial(
        pl.kernel,
        mesh=mesh,
        out_type=jax.ShapeDtypeStruct((2, N_ACC, 128), jnp.float32),
        scratch_types=[
            pltpu.VMEM((40, ACHUNK), jnp.int32),
            pltpu.VMEM((40, ACHUNK), jnp.int32),
            pltpu.VMEM((ACHUNK, 128), jnp.float32),
            pltpu.VMEM((ACHUNK, 128), jnp.float32),
            pltpu.VMEM((ACHUNK, 128), jnp.float32),
            pltpu.VMEM((ACHUNK, 128), jnp.float32),
            pltpu.VMEM_SHARED((N_ACC, 128), jnp.float32),
            pltpu.SemaphoreType.DMA,
            pltpu.SemaphoreType.DMA,
        ],
    )(_agg1_body)
    agg2 = functools.partial(
        pl.kernel,
        mesh=mesh,
        out_type=jax.ShapeDtypeStruct((2, N_ACC, 128), jnp.float32),
        scratch_types=[
            pltpu.VMEM((40, ACHUNK), jnp.int32),
            pltpu.VMEM((40, ACHUNK), jnp.int32),
            pltpu.VMEM((ACHUNK, 128), jnp.float32),
            pltpu.VMEM((ACHUNK, 128), jnp.float32),
            pltpu.VMEM((ACHUNK, 128), jnp.float32),
            pltpu.VMEM((ACHUNK, 128), jnp.float32),
            pltpu.VMEM_SHARED((N_ACC, 128), jnp.float32),
            pltpu.SemaphoreType.DMA,
            pltpu.SemaphoreType.DMA,
        ],
    )(_agg2_body)
    return deg, agg1, agg2


def _deg_body(srcd_hbm, dstp_hbm, ones_hbm, z128_hbm, deg_hbm,
              idx_v, ones_v, acc, sem):
    c = lax.axis_index("c")
    s = lax.axis_index("s")
    rz = s * ROWS_T
    pltpu.sync_copy(z128_hbm.at[pl.ds(rz, ROWS_T)], acc.at[pl.ds(rz, ROWS_T)])
    pltpu.sync_copy(ones_hbm, ones_v)

    @pl.when(c == 0)
    def _():
        pltpu.sync_copy(srcd_hbm.at[s], idx_v)

    @pl.when(c == 1)
    def _():
        pltpu.sync_copy(dstp_hbm.at[s], idx_v)

    plsc.subcore_barrier()

    def body(j, carry):
        @pl.when(j >= 8)
        def _():
            pltpu.make_async_copy(ones_v, acc.at[idx_v.at[0]], sem).wait()

        pltpu.async_copy(ones_v, acc.at[idx_v.at[j]], sem, add=True)
        return carry

    lax.fori_loop(0, 80, body, 0)
    for _ in range(8):
        pltpu.make_async_copy(ones_v, acc.at[idx_v.at[0]], sem).wait()
    plsc.subcore_barrier()

    @pl.when(c == 0)
    def _():
        pltpu.sync_copy(acc.at[pl.ds(rz, ROWS_T)],
                        deg_hbm.at[0, pl.ds(rz, ROWS_T)])

    @pl.when(c == 1)
    def _():
        pltpu.sync_copy(acc.at[pl.ds(rz, ROWS_T)],
                        deg_hbm.at[1, pl.ds(rz, ROWS_T)])


def _agg_pipeline(y_hbm, sidx, didx, bufs, acc, gsem, ssem, nchunks):
    pltpu.async_copy(y_hbm.at[sidx.at[0]], bufs[0], gsem)
    pltpu.async_copy(y_hbm.at[sidx.at[1]], bufs[1], gsem)

    def body(j, carry):
        for b in range(4):
            k = 4 * j + b
            buf = bufs[b]
            nxt = bufs[(b + 2) % 4]
            pltpu.make_async_copy(y_hbm.at[sidx.at[k]], buf, gsem).wait()
            pltpu.async_copy(buf, acc.at[didx.at[k]], ssem, add=True)

            @pl.when(k + 2 < nchunks)
            def _():
                @pl.when(k >= 2)
                def _():
                    pltpu.make_async_copy(nxt, acc.at[didx.at[0]],
                                          ssem).wait()

                pltpu.async_copy(y_hbm.at[sidx.at[k + 2]], nxt, gsem)

        return carry

    lax.fori_loop(0, nchunks // 4, body, 0)
    for _ in range(4):
        pltpu.make_async_copy(bufs[0], acc.at[didx.at[0]], ssem).wait()


def _agg1_body(ya_hbm, yb_hbm, srcg_hbm, dstp_hbm, z128_hbm, out_hbm,
               sidx, didx, b0, b1, b2, b3, acc, gsem, ssem):
    c = lax.axis_index("c")
    s = lax.axis_index("s")
    rz = s * ROWS_T
    pltpu.sync_copy(z128_hbm.at[pl.ds(rz, ROWS_T)], acc.at[pl.ds(rz, ROWS_T)])
    plsc.subcore_barrier()

    def run(y_hbm):
        for h in range(4):
            pltpu.sync_copy(srcg_hbm.at[s, pl.ds(h * 40, 40)], sidx)
            pltpu.sync_copy(dstp_hbm.at[s, pl.ds(h * 40, 40)], didx)
            _agg_pipeline(y_hbm, sidx, didx, (b0, b1, b2, b3), acc,
                          gsem, ssem, 40)

    @pl.when(c == 0)
    def _():
        run(ya_hbm)

    @pl.when(c == 1)
    def _():
        run(yb_hbm)

    plsc.subcore_barrier()

    @pl.when(c == 0)
    def _():
        pltpu.sync_copy(acc.at[pl.ds(rz, ROWS_T)],
                        out_hbm.at[0, pl.ds(rz, ROWS_T)])

    @pl.when(c == 1)
    def _():
        pltpu.sync_copy(acc.at[pl.ds(rz, ROWS_T)],
                        out_hbm.at[1, pl.ds(rz, ROWS_T)])


def _agg2_body(y2_hbm, srcg_hbm, dstp_hbm, z128_hbm, out_hbm,
               sidx, didx, b0, b1, b2, b3, acc, gsem, ssem):
    c = lax.axis_index("c")
    s = lax.axis_index("s")
    rz = s * ROWS_T
    pltpu.sync_copy(z128_hbm.at[pl.ds(rz, ROWS_T)], acc.at[pl.ds(rz, ROWS_T)])

    plsc.subcore_barrier()

    def run(ci):
        for h in range(2):
            pltpu.sync_copy(srcg_hbm.at[ci, s, pl.ds(h * 40, 40)], sidx)
            pltpu.sync_copy(dstp_hbm.at[ci, s, pl.ds(h * 40, 40)], didx)
            _agg_pipeline(y2_hbm, sidx, didx, (b0, b1, b2, b3), acc,
                          gsem, ssem, 40)

    @pl.when(c == 0)
    def _():
        run(0)

    @pl.when(c == 1)
    def _():
        run(1)

    plsc.subcore_barrier()

    @pl.when(c == 0)
    def _():
        pltpu.sync_copy(acc.at[pl.ds(rz, ROWS_T)],
                        out_hbm.at[0, pl.ds(rz, ROWS_T)])

    @pl.when(c == 1)
    def _():
        pltpu.sync_copy(acc.at[pl.ds(rz, ROWS_T)],
                        out_hbm.at[1, pl.ds(rz, ROWS_T)])


def _tc1_body(x_ref, w_ref, deg_ref, ya_ref, yb_ref):
    ns = lax.rsqrt(jnp.maximum(deg_ref[0, :, 0], 1.0))
    y = jnp.dot(x_ref[...], w_ref[...], preferred_element_type=jnp.float32)
    y = y * ns[:, None]
    ya_ref[...] = y[:, :128]
    yb_ref[...] = y[:, 128:]


def _tc2_body(agg_ref, deg_ref, b1_ref, w2_ref, y2_ref):
    a1 = jnp.concatenate([agg_ref[0], agg_ref[1]], axis=1)
    ns = lax.rsqrt(jnp.maximum(deg_ref[0, :, 0], 1.0))
    nd = lax.rsqrt(jnp.maximum(deg_ref[1, :, 0], 1.0))
    g = jnp.maximum(a1 * nd[:, None] + b1_ref[0][None, :], 0.0) * ns[:, None]
    y2 = jnp.dot(g, w2_ref[...], preferred_element_type=jnp.float32)
    y2_ref[...] = jnp.concatenate(
        [y2, jnp.zeros((N_NODES, 128 - N_CLS), jnp.float32)], axis=1)


def _tc3_body(p_ref, deg_ref, b2_ref, out_ref):
    nd = lax.rsqrt(jnp.maximum(deg_ref[1, :, 0], 1.0))
    p = p_ref[0, :, :N_CLS] + p_ref[1, :, :N_CLS]
    out_ref[...] = p * nd[:, None] + b2_ref[0][None, :]


def kernel(x, edge_index, W1, b1, W2, b2):
    src = edge_index[0].astype(jnp.int32)
    dst = edge_index[1].astype(jnp.int32)
    npad = E_PAD - N_EDGES
    src_g = jnp.concatenate(
        [src, jnp.arange(npad, dtype=jnp.int32) % N_NODES])
    dummy = N_NODES + (jnp.arange(npad, dtype=jnp.int32) % (N_ACC - N_NODES))
    src_d = jnp.concatenate([src, dummy])
    dst_p = jnp.concatenate([dst, dummy])

    srcd3 = src_d.reshape(16, 80, CHUNK)
    dstp3 = dst_p.reshape(16, 80, CHUNK)
    srcg3 = src_g.reshape(16, 160, ACHUNK)
    dstp3a = dst_p.reshape(16, 160, ACHUNK)
    srcg4 = src_g.reshape(2, 16, 80, ACHUNK)
    dstp4 = dst_p.reshape(2, 16, 80, ACHUNK)

    ones128 = jnp.zeros((CHUNK, 128), jnp.float32).at[:, 0].set(1.0)
    z128 = jnp.zeros((N_ACC, 128), jnp.float32)

    deg_kernel, agg1_kernel, agg2_kernel = _sc_kernels()
    deg = deg_kernel(srcd3, dstp3, ones128, z128)

    ya, yb = pl.pallas_call(
        _tc1_body,
        grid=(1,),
        in_specs=[
            pl.BlockSpec((N_NODES, D_IN), lambda i: (0, 0)),
            pl.BlockSpec((D_IN, D_HID), lambda i: (0, 0)),
            pl.BlockSpec((2, N_NODES, 128), lambda i: (0, 0, 0)),
        ],
        out_specs=[
            pl.BlockSpec((N_NODES, 128), lambda i: (0, 0)),
            pl.BlockSpec((N_NODES, 128), lambda i: (0, 0)),
        ],
        out_shape=[
            jax.ShapeDtypeStruct((N_NODES, 128), jnp.float32),
            jax.ShapeDtypeStruct((N_NODES, 128), jnp.float32),
        ],
    )(x, W1, deg)

    agg1 = agg1_kernel(ya, yb, srcg3, dstp3a, z128)

    y2 = pl.pallas_call(
        _tc2_body,
        grid=(1,),
        in_specs=[
            pl.BlockSpec((2, N_NODES, 128), lambda i: (0, 0, 0)),
            pl.BlockSpec((2, N_NODES, 128), lambda i: (0, 0, 0)),
            pl.BlockSpec((1, D_HID), lambda i: (0, 0)),
            pl.BlockSpec((D_HID, N_CLS), lambda i: (0, 0)),
        ],
        out_specs=pl.BlockSpec((N_NODES, 128), lambda i: (0, 0)),
        out_shape=jax.ShapeDtypeStruct((N_NODES, 128), jnp.float32),
    )(agg1, deg, b1.reshape(1, D_HID), W2)

    p2 = agg2_kernel(y2, srcg4, dstp4, z128)

    out = pl.pallas_call(
        _tc3_body,
        grid=(1,),
        in_specs=[
            pl.BlockSpec((2, N_NODES, 128), lambda i: (0, 0, 0)),
            pl.BlockSpec((2, N_NODES, 128), lambda i: (0, 0, 0)),
            pl.BlockSpec((1, N_CLS), lambda i: (0, 0)),
        ],
        out_specs=pl.BlockSpec((N_NODES, N_CLS), lambda i: (0, 0)),
        out_shape=jax.ShapeDtypeStruct((N_NODES, N_CLS), jnp.float32),
    )(p2, deg, b2.reshape(1, N_CLS))

    return out
